# baseline (speedup 1.0000x reference)
import functools

import jax
import jax.numpy as jnp
from jax import lax
from jax.experimental import pallas as pl
from jax.experimental.pallas import tpu as pltpu
from jax.experimental.pallas import tpu_sc as plsc

NC = 2
NS = 16
NW = NC * NS

IDX_PER_GATHER = 128
GATHERS_PER_CHUNK = 4
CHUNK = IDX_PER_GATHER * GATHERS_PER_CHUNK


@functools.lru_cache(maxsize=None)
def _make_gather(total, dim):
    per_w = total // NW
    chunks = per_w // CHUNK
    k = GATHERS_PER_CHUNK
    mesh = plsc.VectorSubcoreMesh(core_axis_name="c", subcore_axis_name="s")

    @functools.partial(
        pl.kernel,
        mesh=mesh,
        out_type=jax.ShapeDtypeStruct((total, dim), jnp.float32),
        scratch_types=[
            pltpu.VMEM((k, IDX_PER_GATHER), jnp.int32),
            pltpu.VMEM((CHUNK, dim), jnp.float32),
            pltpu.SemaphoreType.DMA,
        ],
        compiler_params=pltpu.CompilerParams(use_tc_tiling_on_sc=False),
    )
    def grab(table_hbm, idx_hbm, out_hbm, idx_v, rows_v, sem):
        wid = lax.axis_index("s") * NC + lax.axis_index("c")

        def body(g, carry):
            row0 = wid * (per_w // IDX_PER_GATHER) + g * k
            pltpu.sync_copy(idx_hbm.at[pl.ds(row0, k), :], idx_v)
            copies = [
                pltpu.async_copy(
                    table_hbm.at[idx_v.at[j]],
                    rows_v.at[pl.ds(j * IDX_PER_GATHER, IDX_PER_GATHER), :],
                    sem,
                )
                for j in range(k)
            ]
            for c in copies:
                c.wait()
            base = wid * per_w + g * CHUNK
            pltpu.sync_copy(rows_v, out_hbm.at[pl.ds(base, CHUNK), :])
            return carry

        lax.fori_loop(0, chunks, body, 0)

    return grab


def kernel(text, embedding_weight):
    b, l = text.shape
    total = b * l
    dim = embedding_weight.shape[1]
    idx2d = text.reshape(total // IDX_PER_GATHER, IDX_PER_GATHER).astype(jnp.int32)
    out = _make_gather(total, dim)(embedding_weight, idx2d)
    return out.reshape(b, l, dim)


# speedup vs baseline: 1.0443x; 1.0443x over previous
import functools

import jax
import jax.numpy as jnp
from jax import lax
from jax.experimental import pallas as pl
from jax.experimental.pallas import tpu as pltpu
from jax.experimental.pallas import tpu_sc as plsc

NC = 2
NS = 16
NW = NC * NS

IDX_PER_GATHER = 128
GATHERS_PER_CHUNK = 4
CHUNK = IDX_PER_GATHER * GATHERS_PER_CHUNK


@functools.lru_cache(maxsize=None)
def _make_gather(total, dim):
    per_w = total // NW
    chunks = per_w // CHUNK
    k = GATHERS_PER_CHUNK
    mesh = plsc.VectorSubcoreMesh(core_axis_name="c", subcore_axis_name="s")

    assert chunks % 2 == 0

    @functools.partial(
        pl.kernel,
        mesh=mesh,
        out_type=jax.ShapeDtypeStruct((total, dim), jnp.float32),
        scratch_types=[
            pltpu.VMEM((2, k, IDX_PER_GATHER), jnp.int32),
            pltpu.VMEM((2, CHUNK, dim), jnp.float32),
            pltpu.SemaphoreType.DMA,
            pltpu.SemaphoreType.DMA,
            pltpu.SemaphoreType.DMA,
            pltpu.SemaphoreType.DMA,
        ],
        compiler_params=pltpu.CompilerParams(use_tc_tiling_on_sc=False),
    )
    def grab(table_hbm, idx_hbm, out_hbm, idx_v, rows_v, g0, g1, w0, w1):
        wid = lax.axis_index("s") * NC + lax.axis_index("c")
        gsems = (g0, g1)
        wsems = (w0, w1)

        def fire_chunk(g, b):
            row0 = wid * (per_w // IDX_PER_GATHER) + g * k
            pltpu.sync_copy(idx_hbm.at[pl.ds(row0, k), :], idx_v.at[b])
            for j in range(k):
                pltpu.async_copy(
                    table_hbm.at[idx_v.at[b, j]],
                    rows_v.at[b, pl.ds(j * IDX_PER_GATHER, IDX_PER_GATHER), :],
                    gsems[b],
                )

        def drain_gathers(b):
            for j in range(k):
                pltpu.make_async_copy(
                    table_hbm.at[idx_v.at[b, j]],
                    rows_v.at[b, pl.ds(j * IDX_PER_GATHER, IDX_PER_GATHER), :],
                    gsems[b],
                ).wait()

        def out_slice(g):
            base = wid * per_w + g * CHUNK
            return out_hbm.at[pl.ds(base, CHUNK), :]

        def body(outer, carry):
            gA = 2 * outer

            for b in range(2):
                @pl.when(outer > 0)
                def _():
                    pltpu.make_async_copy(rows_v.at[b], out_slice(gA + b),
                                          wsems[b]).wait()
                fire_chunk(gA + b, b)

            for b in range(2):
                drain_gathers(b)
                pltpu.async_copy(rows_v.at[b], out_slice(gA + b), wsems[b])

            return carry

        lax.fori_loop(0, chunks // 2, body, 0)
        last = chunks - 2
        for b in range(2):
            pltpu.make_async_copy(rows_v.at[b], out_slice(last + b),
                                  wsems[b]).wait()

    return grab


def kernel(text, embedding_weight):
    b, l = text.shape
    total = b * l
    dim = embedding_weight.shape[1]
    idx2d = text.reshape(total // IDX_PER_GATHER, IDX_PER_GATHER).astype(jnp.int32)
    out = _make_gather(total, dim)(embedding_weight, idx2d)
    return out.reshape(b, l, dim)
